# Initial kernel scaffold; baseline (speedup 1.0000x reference)
import jax
import jax.numpy as jnp
from jax.experimental import pallas as pl


def kernel(edge_index, features, edge_weights, W, b):
    raise NotImplementedError("write your pallas kernel here")


# speedup vs baseline: 4.6994x; 4.6994x over previous
import functools

import jax
import jax.numpy as jnp
from jax import lax
from jax.experimental import pallas as pl
from jax.experimental.pallas import tpu as pltpu
from jax.experimental.pallas import tpu_sc as plsc

N = 10000
E = 320000
D = 128

CHUNK = 128
NCHUNK = E // CHUNK
NWORKERS = 32
NPAD = 10240
ROWS_PER_TILE = NPAD // 16


def _sc_propagate_body(src_hbm, dst_hbm, w_hbm, feat_hbm, out_hbm,
                       src_v, dst_v, w_v, rows, acc, sem):
    cid = lax.axis_index("c")
    sid = lax.axis_index("s")
    wid = sid * 2 + cid

    zeros = jnp.zeros((16,), jnp.float32)

    def zbody(i, carry):
        for d in range(D // 16):
            rows[i, pl.ds(d * 16, 16)] = zeros
        return carry

    lax.fori_loop(0, CHUNK, zbody, 0)

    tile_base = sid * ROWS_PER_TILE
    for j in range(ROWS_PER_TILE // CHUNK):
        pltpu.sync_copy(rows, acc.at[pl.ds(tile_base + j * CHUNK, CHUNK)])

    plsc.subcore_barrier()

    trip = (NCHUNK - wid + NWORKERS - 1) // NWORKERS

    def ebody(k, carry):
        base = (wid + k * NWORKERS) * CHUNK
        pltpu.sync_copy(src_hbm.at[pl.ds(base, CHUNK)], src_v)
        pltpu.sync_copy(dst_hbm.at[pl.ds(base, CHUNK)], dst_v)
        pltpu.sync_copy(w_hbm.at[pl.ds(base, CHUNK)], w_v)
        pltpu.async_copy(feat_hbm.at[src_v], rows, sem).wait()

        def sbody(e, c2):
            widx = jnp.full((16,), e, jnp.int32)
            wv = plsc.load_gather(w_v, [widx])
            for d in range(D // 16):
                sl = pl.ds(d * 16, 16)
                rows[e, sl] = rows[e, sl] * wv
            return c2

        lax.fori_loop(0, CHUNK, sbody, 0)

        pltpu.sync_copy(rows, acc.at[dst_v], add=True)
        return carry

    lax.fori_loop(0, trip, ebody, 0)

    plsc.subcore_barrier()

    pltpu.sync_copy(acc.at[pl.ds(tile_base, ROWS_PER_TILE)],
                    out_hbm.at[cid, pl.ds(tile_base, ROWS_PER_TILE)])


_sc_propagate = functools.partial(
    pl.kernel,
    out_type=jax.ShapeDtypeStruct((2, NPAD, D), jnp.float32),
    mesh=plsc.VectorSubcoreMesh(core_axis_name="c", subcore_axis_name="s"),
    scratch_types=[
        pltpu.VMEM((CHUNK,), jnp.int32),
        pltpu.VMEM((CHUNK,), jnp.int32),
        pltpu.VMEM((CHUNK,), jnp.float32),
        pltpu.VMEM((CHUNK, D), jnp.float32),
        pltpu.VMEM_SHARED((NPAD, D), jnp.float32),
        pltpu.SemaphoreType.DMA,
    ],
    compiler_params=pltpu.CompilerParams(needs_layout_passes=False),
)(_sc_propagate_body)


ROW_BLK = 400


def _tc_finish_body(a0_ref, a1_ref, w_ref, b_ref, o_ref):
    z = jnp.dot(a0_ref[...] + a1_ref[...], w_ref[...],
                preferred_element_type=jnp.float32)
    z = z + b_ref[...]
    m = jnp.max(z, axis=1, keepdims=True)
    ez = jnp.exp(z - m)
    s = jnp.sum(ez, axis=1, keepdims=True)
    o_ref[...] = z - m - jnp.log(s)


def _tc_finish(a0, a1, W, b2d):
    return pl.pallas_call(
        _tc_finish_body,
        grid=(N // ROW_BLK,),
        in_specs=[
            pl.BlockSpec((ROW_BLK, D), lambda i: (i, 0)),
            pl.BlockSpec((ROW_BLK, D), lambda i: (i, 0)),
            pl.BlockSpec((D, D), lambda i: (0, 0)),
            pl.BlockSpec((1, D), lambda i: (0, 0)),
        ],
        out_specs=pl.BlockSpec((ROW_BLK, D), lambda i: (i, 0)),
        out_shape=jax.ShapeDtypeStruct((N, D), jnp.float32),
    )(a0, a1, W, b2d)


def kernel(edge_index, features, edge_weights, W, b):
    src = edge_index[0]
    dst = edge_index[1]
    partials = _sc_propagate(src, dst, edge_weights, features)
    return _tc_finish(partials[0, :N], partials[1, :N], W, b.reshape(1, D))
